# baseline (speedup 1.0000x reference)
import numpy as np
import jax
import jax.numpy as jnp
from jax.experimental import pallas as pl
from jax.experimental.pallas import tpu as pltpu

_TIME_MASK_PARAM = 100
_FREQ_MASK_PARAM = 27
_NUM_TIME_MASKS = 2
_NUM_FREQ_MASKS = 2
_MASK_VALUE = 0.0

_T_BLK = 3000
_MASKS_PER_SAMPLE = _NUM_FREQ_MASKS + _NUM_TIME_MASKS
_STATS_COLS = 3 * _MASKS_PER_SAMPLE + 1


def _threefry2x32(k0, k1, x0, x1):
    M = 0xFFFFFFFF
    rot = ((13, 15, 26, 6), (17, 29, 16, 24))
    ks = (k0, k1, (k0 ^ k1 ^ 0x1BD11BDA) & M)
    x0 = (x0 + ks[0]) & M
    x1 = (x1 + ks[1]) & M
    for i in range(5):
        for r in rot[i % 2]:
            x0 = (x0 + x1) & M
            x1 = ((x1 << r) | (x1 >> (32 - r))) & M
            x1 ^= x0
        x0 = (x0 + ks[(i + 1) % 3]) & M
        x1 = (x1 + ks[(i + 2) % 3] + i + 1) & M
    return x0, x1


def _build_draw_table(n):
    k = (0, 42)
    tab = np.zeros((n, 4), dtype=np.int32)
    for j in range(n):
        d = _threefry2x32(k[0], k[1], 0, 1)
        p = _threefry2x32(d[0], d[1], 0, 0)
        r = _threefry2x32(d[0], d[1], 0, 1)
        ps = _threefry2x32(p[0], p[1], 0, 0)
        rs = _threefry2x32(r[0], r[1], 0, 0)
        s = ps[0] ^ ps[1]
        t = rs[0] ^ rs[1]
        tab[j] = (s >> 16, s & 0xFFFF, t >> 16, t & 0xFFFF)
        k = _threefry2x32(k[0], k[1], 0, 0)
    return tab


_TAB = _build_draw_table(2 * _MASKS_PER_SAMPLE * 64 + 8)


def _resolve_body(len_ref, tab_ref, out_ref):
    B = len_ref.shape[0]
    F = 80

    def draw(a, span):
        m16 = 65536 % span
        m32 = (m16 * m16) % span
        s_m = ((tab_ref[a, 0] % span) * m16 + (tab_ref[a, 1] % span)) % span
        t_m = ((tab_ref[a, 2] % span) * m16 + (tab_ref[a, 3] % span)) % span
        return (s_m * m32 + t_m) % span

    def body(b, a):
        L = len_ref[b]
        valid = jnp.where(L > 0, 1, 0).astype(jnp.int32)
        for i in range(_NUM_FREQ_MASKS):
            w = draw(a, min(_FREQ_MASK_PARAM, F) + 1)
            a1 = a + valid
            c = valid * jnp.where((w > 0) & (w < F), 1, 0)
            s = draw(a1, F - w + 1)
            out_ref[b, 3 * i + 0] = s
            out_ref[b, 3 * i + 1] = w
            out_ref[b, 3 * i + 2] = c
            a = a1 + c
        for i in range(_NUM_TIME_MASKS):
            w = draw(a, jnp.minimum(_TIME_MASK_PARAM, L) + 1)
            a1 = a + valid
            c = valid * jnp.where((w > 0) & (w < L), 1, 0)
            s = draw(a1, L - w + 1)
            j = 3 * (_NUM_FREQ_MASKS + i)
            out_ref[b, j + 0] = s
            out_ref[b, j + 1] = w
            out_ref[b, j + 2] = c
            a = a1 + c
        out_ref[b, _STATS_COLS - 1] = L
        return a

    jax.lax.fori_loop(0, B, body, jnp.int32(0))


_F = 80
_FOLD = 8
_ROWS = 3000 // _FOLD


def _apply_body(stats_ref, mel_ref, out_ref):
    b = pl.program_id(0)

    fs0 = stats_ref[b, 0]
    fw0 = stats_ref[b, 1]
    fc0 = stats_ref[b, 2]
    fs1 = stats_ref[b, 3]
    fw1 = stats_ref[b, 4]
    fc1 = stats_ref[b, 5]
    ts0 = stats_ref[b, 6]
    tw0 = stats_ref[b, 7]
    tc0 = stats_ref[b, 8]
    ts1 = stats_ref[b, 9]
    tw1 = stats_ref[b, 10]
    tc1 = stats_ref[b, 11]
    length = stats_ref[b, 12]

    shape = (_ROWS, _FOLD * _F)
    rr = jax.lax.broadcasted_iota(jnp.int32, shape, 0)
    jj = jax.lax.broadcasted_iota(jnp.int32, shape, 1)
    q = jnp.zeros(shape, jnp.int32)
    for k in range(1, _FOLD):
        q = q + (jj >= k * _F).astype(jnp.int32)
    tt = rr * _FOLD + q
    ff = jj - q * _F

    fmask = ((fc0 > 0) & (ff >= fs0) & (ff < fs0 + fw0)) | (
        (fc1 > 0) & (ff >= fs1) & (ff < fs1 + fw1)
    )
    fmask = fmask & (tt < length)
    tmask = ((tc0 > 0) & (tt >= ts0) & (tt < ts0 + tw0)) | (
        (tc1 > 0) & (tt >= ts1) & (tt < ts1 + tw1)
    )
    mask = fmask | tmask
    out_ref[0] = jnp.where(mask, jnp.float32(_MASK_VALUE), mel_ref[0])


def kernel(mel, lengths):
    B, T, F = mel.shape
    stats = pl.pallas_call(
        _resolve_body,
        in_specs=[
            pl.BlockSpec(memory_space=pltpu.SMEM),
            pl.BlockSpec(memory_space=pltpu.SMEM),
        ],
        out_specs=pl.BlockSpec(memory_space=pltpu.SMEM),
        out_shape=jax.ShapeDtypeStruct((B, _STATS_COLS), jnp.int32),
    )(lengths.astype(jnp.int32), jnp.asarray(_TAB))
    mel_folded = jnp.reshape(mel, (B, _ROWS, _FOLD * _F))
    out = pl.pallas_call(
        _apply_body,
        grid=(B,),
        in_specs=[
            pl.BlockSpec(memory_space=pltpu.SMEM),
            pl.BlockSpec((1, _ROWS, _FOLD * _F), lambda b: (b, 0, 0)),
        ],
        out_specs=pl.BlockSpec((1, _ROWS, _FOLD * _F), lambda b: (b, 0, 0)),
        out_shape=jax.ShapeDtypeStruct((B, _ROWS, _FOLD * _F), mel.dtype),
        compiler_params=pltpu.CompilerParams(
            dimension_semantics=("parallel",),
        ),
    )(stats, mel_folded)
    return jnp.reshape(out, (B, T, F))


# speedup vs baseline: 2.0697x; 2.0697x over previous
import numpy as np
import jax
import jax.numpy as jnp
from jax.experimental import pallas as pl
from jax.experimental.pallas import tpu as pltpu

_TIME_MASK_PARAM = 100
_FREQ_MASK_PARAM = 27
_NUM_TIME_MASKS = 2
_NUM_FREQ_MASKS = 2
_MASK_VALUE = 0.0

_S_BLK = 8
_MASKS_PER_SAMPLE = _NUM_FREQ_MASKS + _NUM_TIME_MASKS
_STATS_COLS = 3 * _MASKS_PER_SAMPLE + 1


def _threefry2x32(k0, k1, x0, x1):
    M = 0xFFFFFFFF
    rot = ((13, 15, 26, 6), (17, 29, 16, 24))
    ks = (k0, k1, (k0 ^ k1 ^ 0x1BD11BDA) & M)
    x0 = (x0 + ks[0]) & M
    x1 = (x1 + ks[1]) & M
    for i in range(5):
        for r in rot[i % 2]:
            x0 = (x0 + x1) & M
            x1 = ((x1 << r) | (x1 >> (32 - r))) & M
            x1 ^= x0
        x0 = (x0 + ks[(i + 1) % 3]) & M
        x1 = (x1 + ks[(i + 2) % 3] + i + 1) & M
    return x0, x1


def _build_draw_table(n):
    k = (0, 42)
    tab = np.zeros((n, 4), dtype=np.int32)
    for j in range(n):
        d = _threefry2x32(k[0], k[1], 0, 1)
        p = _threefry2x32(d[0], d[1], 0, 0)
        r = _threefry2x32(d[0], d[1], 0, 1)
        ps = _threefry2x32(p[0], p[1], 0, 0)
        rs = _threefry2x32(r[0], r[1], 0, 0)
        s = ps[0] ^ ps[1]
        t = rs[0] ^ rs[1]
        tab[j] = (s >> 16, s & 0xFFFF, t >> 16, t & 0xFFFF)
        k = _threefry2x32(k[0], k[1], 0, 0)
    return tab


_TAB = _build_draw_table(2 * _MASKS_PER_SAMPLE * 64 + 8)


def _resolve(len_ref, tab_ref, stats_ref, B, F):
    def draw(a, span):
        m16 = 65536 % span
        m32 = (m16 * m16) % span
        s_m = ((tab_ref[a, 0] % span) * m16 + (tab_ref[a, 1] % span)) % span
        t_m = ((tab_ref[a, 2] % span) * m16 + (tab_ref[a, 3] % span)) % span
        return (s_m * m32 + t_m) % span

    def body(b, a):
        L = len_ref[b]
        valid = jnp.where(L > 0, 1, 0).astype(jnp.int32)
        for i in range(_NUM_FREQ_MASKS):
            w = draw(a, min(_FREQ_MASK_PARAM, F) + 1)
            a1 = a + valid
            c = valid * jnp.where((w > 0) & (w < F), 1, 0)
            s = draw(a1, F - w + 1)
            stats_ref[b, 3 * i + 0] = s
            stats_ref[b, 3 * i + 1] = w
            stats_ref[b, 3 * i + 2] = c
            a = a1 + c
        for i in range(_NUM_TIME_MASKS):
            w = draw(a, jnp.minimum(_TIME_MASK_PARAM, L) + 1)
            a1 = a + valid
            c = valid * jnp.where((w > 0) & (w < L), 1, 0)
            s = draw(a1, L - w + 1)
            j = 3 * (_NUM_FREQ_MASKS + i)
            stats_ref[b, j + 0] = s
            stats_ref[b, j + 1] = w
            stats_ref[b, j + 2] = c
            a = a1 + c
        stats_ref[b, _STATS_COLS - 1] = L
        return a

    jax.lax.fori_loop(0, B, body, jnp.int32(0))


def _fused_body(len_ref, tab_ref, mel_ref, out_ref, stats_ref):
    g = pl.program_id(0)
    B = len_ref.shape[0]
    T, F = mel_ref.shape[1], mel_ref.shape[2]

    @pl.when(g == 0)
    def _():
        _resolve(len_ref, tab_ref, stats_ref, B, F)

    tt = jax.lax.broadcasted_iota(jnp.int32, (T, F), 0)
    ff = jax.lax.broadcasted_iota(jnp.int32, (T, F), 1)

    for s in range(_S_BLK):
        b = g * _S_BLK + s
        fs0 = stats_ref[b, 0]
        fw0 = stats_ref[b, 1]
        fc0 = stats_ref[b, 2]
        fs1 = stats_ref[b, 3]
        fw1 = stats_ref[b, 4]
        fc1 = stats_ref[b, 5]
        ts0 = stats_ref[b, 6]
        tw0 = stats_ref[b, 7]
        tc0 = stats_ref[b, 8]
        ts1 = stats_ref[b, 9]
        tw1 = stats_ref[b, 10]
        tc1 = stats_ref[b, 11]
        length = stats_ref[b, 12]

        fmask = ((fc0 > 0) & (ff >= fs0) & (ff < fs0 + fw0)) | (
            (fc1 > 0) & (ff >= fs1) & (ff < fs1 + fw1)
        )
        fmask = fmask & (tt < length)
        tmask = ((tc0 > 0) & (tt >= ts0) & (tt < ts0 + tw0)) | (
            (tc1 > 0) & (tt >= ts1) & (tt < ts1 + tw1)
        )
        mask = fmask | tmask
        out_ref[s] = jnp.where(mask, jnp.float32(_MASK_VALUE), mel_ref[s])


def kernel(mel, lengths):
    B, T, F = mel.shape
    out = pl.pallas_call(
        _fused_body,
        grid=(B // _S_BLK,),
        in_specs=[
            pl.BlockSpec(memory_space=pltpu.SMEM),
            pl.BlockSpec(memory_space=pltpu.SMEM),
            pl.BlockSpec((_S_BLK, T, F), lambda b: (b, 0, 0)),
        ],
        out_specs=pl.BlockSpec((_S_BLK, T, F), lambda b: (b, 0, 0)),
        out_shape=jax.ShapeDtypeStruct((B, T, F), mel.dtype),
        scratch_shapes=[pltpu.SMEM((B, _STATS_COLS), jnp.int32)],
        compiler_params=pltpu.CompilerParams(
            dimension_semantics=("arbitrary",),
        ),
    )(lengths.astype(jnp.int32), jnp.asarray(_TAB), mel)
    return out


# speedup vs baseline: 2.1700x; 1.0484x over previous
import numpy as np
import jax
import jax.numpy as jnp
from jax.experimental import pallas as pl
from jax.experimental.pallas import tpu as pltpu

_TIME_MASK_PARAM = 100
_FREQ_MASK_PARAM = 27
_NUM_TIME_MASKS = 2
_NUM_FREQ_MASKS = 2
_MASK_VALUE = 0.0

_S_BLK = 8
_MASKS_PER_SAMPLE = _NUM_FREQ_MASKS + _NUM_TIME_MASKS
_STATS_COLS = 3 * _MASKS_PER_SAMPLE + 1


def _threefry2x32(k0, k1, x0, x1):
    M = 0xFFFFFFFF
    rot = ((13, 15, 26, 6), (17, 29, 16, 24))
    ks = (k0, k1, (k0 ^ k1 ^ 0x1BD11BDA) & M)
    x0 = (x0 + ks[0]) & M
    x1 = (x1 + ks[1]) & M
    for i in range(5):
        for r in rot[i % 2]:
            x0 = (x0 + x1) & M
            x1 = ((x1 << r) | (x1 >> (32 - r))) & M
            x1 ^= x0
        x0 = (x0 + ks[(i + 1) % 3]) & M
        x1 = (x1 + ks[(i + 2) % 3] + i + 1) & M
    return x0, x1


def _randint_from(s, t, span):
    return ((s % span) * ((2**32) % span) + (t % span)) % span


def _build_draw_table(n):
    k = (0, 42)
    raw = np.zeros((n + 1, 2), dtype=np.uint64)
    for j in range(n + 1):
        d = _threefry2x32(k[0], k[1], 0, 1)
        p = _threefry2x32(d[0], d[1], 0, 0)
        r = _threefry2x32(d[0], d[1], 0, 1)
        ps = _threefry2x32(p[0], p[1], 0, 0)
        rs = _threefry2x32(r[0], r[1], 0, 0)
        raw[j] = (ps[0] ^ ps[1], rs[0] ^ rs[1])
        k = _threefry2x32(k[0], k[1], 0, 0)
    tab = np.zeros((n, 6), dtype=np.int32)
    for j in range(n):
        s, t = int(raw[j, 0]), int(raw[j, 1])
        w = _randint_from(s, t, _FREQ_MASK_PARAM + 1)
        sf = _randint_from(int(raw[j + 1, 0]), int(raw[j + 1, 1]), 81 - w)
        tab[j] = (s >> 16, s & 0xFFFF, t >> 16, t & 0xFFFF, w, sf)
    return tab


_TAB = _build_draw_table(2 * _MASKS_PER_SAMPLE * 64 + 8)


def _resolve(len_ref, tab_ref, stats_ref, B, F):
    def draw(a, span):
        m16 = 65536 % span
        m32 = (m16 * m16) % span
        s_m = ((tab_ref[a, 0] % span) * m16 + (tab_ref[a, 1] % span)) % span
        t_m = ((tab_ref[a, 2] % span) * m16 + (tab_ref[a, 3] % span)) % span
        return (s_m * m32 + t_m) % span

    def body(b, a):
        L = len_ref[b]
        valid = jnp.where(L > 0, 1, 0).astype(jnp.int32)
        for i in range(_NUM_FREQ_MASKS):
            w = tab_ref[a, 4]
            c = valid * jnp.where((w > 0) & (w < F), 1, 0)
            stats_ref[b, 3 * i + 0] = tab_ref[a, 5]
            stats_ref[b, 3 * i + 1] = w
            stats_ref[b, 3 * i + 2] = c
            a = a + valid + c
        for i in range(_NUM_TIME_MASKS):
            w = draw(a, jnp.minimum(_TIME_MASK_PARAM, L) + 1)
            a1 = a + valid
            c = valid * jnp.where((w > 0) & (w < L), 1, 0)
            s = draw(a1, L - w + 1)
            j = 3 * (_NUM_FREQ_MASKS + i)
            stats_ref[b, j + 0] = s
            stats_ref[b, j + 1] = w
            stats_ref[b, j + 2] = c
            a = a1 + c
        stats_ref[b, _STATS_COLS - 1] = L
        return a

    jax.lax.fori_loop(0, B, body, jnp.int32(0))


def _fused_body(len_ref, tab_ref, mel_ref, out_ref, stats_ref):
    g = pl.program_id(0)
    B = len_ref.shape[0]
    T, F = mel_ref.shape[1], mel_ref.shape[2]

    @pl.when(g == 0)
    def _():
        _resolve(len_ref, tab_ref, stats_ref, B, F)

    tt = jax.lax.broadcasted_iota(jnp.int32, (T, F), 0)
    ff = jax.lax.broadcasted_iota(jnp.int32, (T, F), 1)

    for s in range(_S_BLK):
        b = g * _S_BLK + s
        fs0 = stats_ref[b, 0]
        fw0 = stats_ref[b, 1]
        fc0 = stats_ref[b, 2]
        fs1 = stats_ref[b, 3]
        fw1 = stats_ref[b, 4]
        fc1 = stats_ref[b, 5]
        ts0 = stats_ref[b, 6]
        tw0 = stats_ref[b, 7]
        tc0 = stats_ref[b, 8]
        ts1 = stats_ref[b, 9]
        tw1 = stats_ref[b, 10]
        tc1 = stats_ref[b, 11]
        length = stats_ref[b, 12]

        fmask = ((fc0 > 0) & (ff >= fs0) & (ff < fs0 + fw0)) | (
            (fc1 > 0) & (ff >= fs1) & (ff < fs1 + fw1)
        )
        fmask = fmask & (tt < length)
        tmask = ((tc0 > 0) & (tt >= ts0) & (tt < ts0 + tw0)) | (
            (tc1 > 0) & (tt >= ts1) & (tt < ts1 + tw1)
        )
        mask = fmask | tmask
        out_ref[s] = jnp.where(mask, jnp.float32(_MASK_VALUE), mel_ref[s])


def kernel(mel, lengths):
    B, T, F = mel.shape
    out = pl.pallas_call(
        _fused_body,
        grid=(B // _S_BLK,),
        in_specs=[
            pl.BlockSpec(memory_space=pltpu.SMEM),
            pl.BlockSpec(memory_space=pltpu.SMEM),
            pl.BlockSpec((_S_BLK, T, F), lambda b: (b, 0, 0)),
        ],
        out_specs=pl.BlockSpec((_S_BLK, T, F), lambda b: (b, 0, 0)),
        out_shape=jax.ShapeDtypeStruct((B, T, F), mel.dtype),
        scratch_shapes=[pltpu.SMEM((B, _STATS_COLS), jnp.int32)],
        compiler_params=pltpu.CompilerParams(
            dimension_semantics=("arbitrary",),
        ),
    )(lengths.astype(jnp.int32), jnp.asarray(_TAB), mel)
    return out


# speedup vs baseline: 2.4467x; 1.1275x over previous
import numpy as np
import jax
import jax.numpy as jnp
from jax.experimental import pallas as pl
from jax.experimental.pallas import tpu as pltpu

_TIME_MASK_PARAM = 100
_FREQ_MASK_PARAM = 27
_NUM_TIME_MASKS = 2
_NUM_FREQ_MASKS = 2
_MASK_VALUE = 0.0

_S_BLK = 8
_MASKS_PER_SAMPLE = _NUM_FREQ_MASKS + _NUM_TIME_MASKS
_STATS_COLS = 2 * _MASKS_PER_SAMPLE + 1


def _threefry2x32(k0, k1, x0, x1):
    M = 0xFFFFFFFF
    rot = ((13, 15, 26, 6), (17, 29, 16, 24))
    ks = (k0, k1, (k0 ^ k1 ^ 0x1BD11BDA) & M)
    x0 = (x0 + ks[0]) & M
    x1 = (x1 + ks[1]) & M
    for i in range(5):
        for r in rot[i % 2]:
            x0 = (x0 + x1) & M
            x1 = ((x1 << r) | (x1 >> (32 - r))) & M
            x1 ^= x0
        x0 = (x0 + ks[(i + 1) % 3]) & M
        x1 = (x1 + ks[(i + 2) % 3] + i + 1) & M
    return x0, x1


def _randint_from(s, t, span):
    return ((s % span) * ((2**32) % span) + (t % span)) % span


def _build_draw_table(n):
    k = (0, 42)
    raw = np.zeros((n + 1, 2), dtype=np.uint64)
    for j in range(n + 1):
        d = _threefry2x32(k[0], k[1], 0, 1)
        p = _threefry2x32(d[0], d[1], 0, 0)
        r = _threefry2x32(d[0], d[1], 0, 1)
        ps = _threefry2x32(p[0], p[1], 0, 0)
        rs = _threefry2x32(r[0], r[1], 0, 0)
        raw[j] = (ps[0] ^ ps[1], rs[0] ^ rs[1])
        k = _threefry2x32(k[0], k[1], 0, 0)
    tab = np.zeros((n, 6), dtype=np.int32)
    for j in range(n):
        s, t = int(raw[j, 0]), int(raw[j, 1])
        w = _randint_from(s, t, _FREQ_MASK_PARAM + 1)
        sf = _randint_from(int(raw[j + 1, 0]), int(raw[j + 1, 1]), 81 - w)
        tab[j] = (s >> 16, s & 0xFFFF, t >> 16, t & 0xFFFF, w, sf)
    return tab


_TAB = _build_draw_table(2 * _MASKS_PER_SAMPLE * 64 + 8)


def _resolve(len_ref, tab_ref, stats_ref, B, F):
    def draw(a, span):
        m16 = 65536 % span
        m32 = (m16 * m16) % span
        s_m = ((tab_ref[a, 0] % span) * m16 + (tab_ref[a, 1] % span)) % span
        t_m = ((tab_ref[a, 2] % span) * m16 + (tab_ref[a, 3] % span)) % span
        return (s_m * m32 + t_m) % span

    def body(b, a):
        L = len_ref[b]
        valid = jnp.where(L > 0, 1, 0).astype(jnp.int32)
        for i in range(_NUM_FREQ_MASKS):
            w = tab_ref[a, 4]
            c = valid * jnp.where((w > 0) & (w < F), 1, 0)
            stats_ref[b, 2 * i + 0] = tab_ref[a, 5]
            stats_ref[b, 2 * i + 1] = c * w
            a = a + valid + c
        for i in range(_NUM_TIME_MASKS):
            w = draw(a, jnp.minimum(_TIME_MASK_PARAM, L) + 1)
            a1 = a + valid
            c = valid * jnp.where((w > 0) & (w < L), 1, 0)
            s = draw(a1, L - w + 1)
            j = 2 * (_NUM_FREQ_MASKS + i)
            stats_ref[b, j + 0] = s
            stats_ref[b, j + 1] = c * w
            a = a1 + c
        stats_ref[b, _STATS_COLS - 1] = L
        return a

    jax.lax.fori_loop(0, B, body, jnp.int32(0))


def _fused_body(len_ref, tab_ref, mel_ref, out_ref, stats_ref):
    g = pl.program_id(0)
    B = len_ref.shape[0]
    T, F = mel_ref.shape[1], mel_ref.shape[2]

    @pl.when(g == 0)
    def _():
        _resolve(len_ref, tab_ref, stats_ref, B, F)

    tt = jax.lax.broadcasted_iota(jnp.uint32, (T, F), 0)
    ff = jax.lax.broadcasted_iota(jnp.uint32, (T, F), 1)

    for s in range(_S_BLK):
        b = g * _S_BLK + s
        fs0 = stats_ref[b, 0].astype(jnp.uint32)
        fw0 = stats_ref[b, 1].astype(jnp.uint32)
        fs1 = stats_ref[b, 2].astype(jnp.uint32)
        fw1 = stats_ref[b, 3].astype(jnp.uint32)
        ts0 = stats_ref[b, 4].astype(jnp.uint32)
        tw0 = stats_ref[b, 5].astype(jnp.uint32)
        ts1 = stats_ref[b, 6].astype(jnp.uint32)
        tw1 = stats_ref[b, 7].astype(jnp.uint32)
        length = stats_ref[b, 8].astype(jnp.uint32)

        fmask = ((ff - fs0) < fw0) | ((ff - fs1) < fw1)
        fmask = fmask & (tt < length)
        tmask = ((tt - ts0) < tw0) | ((tt - ts1) < tw1)
        mask = fmask | tmask
        out_ref[s] = jnp.where(mask, jnp.float32(_MASK_VALUE), mel_ref[s])


def kernel(mel, lengths):
    B, T, F = mel.shape
    out = pl.pallas_call(
        _fused_body,
        grid=(B // _S_BLK,),
        in_specs=[
            pl.BlockSpec(memory_space=pltpu.SMEM),
            pl.BlockSpec(memory_space=pltpu.SMEM),
            pl.BlockSpec((_S_BLK, T, F), lambda b: (b, 0, 0)),
        ],
        out_specs=pl.BlockSpec((_S_BLK, T, F), lambda b: (b, 0, 0)),
        out_shape=jax.ShapeDtypeStruct((B, T, F), mel.dtype),
        scratch_shapes=[pltpu.SMEM((B, _STATS_COLS), jnp.int32)],
        compiler_params=pltpu.CompilerParams(
            dimension_semantics=("arbitrary",),
        ),
    )(lengths.astype(jnp.int32), jnp.asarray(_TAB), mel)
    return out
